# Initial kernel scaffold; baseline (speedup 1.0000x reference)
import jax
import jax.numpy as jnp
from jax.experimental import pallas as pl


def kernel(x, edge_index, W1, b1, W2, b2, Wres, bres, Wout, bout):
    raise NotImplementedError("write your pallas kernel here")


# speedup vs baseline: 12.8269x; 12.8269x over previous
import functools

import jax
import jax.numpy as jnp
from jax import lax
from jax.experimental import pallas as pl
from jax.experimental.pallas import tpu as pltpu
from jax.experimental.pallas import tpu_sc as plsc

N = 10000
NP = 10240
E = 320000
D = 128
NCORES = 2
NSUB = 16
NW = NCORES * NSUB
EPT = E // NW
CHUNK = 112
NCH = 91
PADE = NCH * CHUNK
PADV = NP - 1
RPT = NP // NSUB
BR = 1024


def _sc_degree():
    mesh = plsc.VectorSubcoreMesh(core_axis_name="c", subcore_axis_name="s")

    @functools.partial(
        pl.kernel,
        compiler_params=pltpu.CompilerParams(use_tc_tiling_on_sc=False),
        out_type=jax.ShapeDtypeStruct((NCORES * NP,), jnp.float32),
        mesh=mesh,
        scratch_types=[
            pltpu.VMEM_SHARED((NP,), jnp.float32),
            pltpu.VMEM((NCH, CHUNK), jnp.int32),
            pltpu.VMEM((RPT,), jnp.float32),
            pltpu.VMEM((CHUNK,), jnp.float32),
        ],
    )
    def deg_kernel(eidx_hbm, out_hbm, acc, idx_d, zbuf, ones):
        c = lax.axis_index("c")
        s = lax.axis_index("s")
        wid = s * NCORES + c
        zero16 = jnp.zeros((16,), jnp.float32)
        one16 = jnp.ones((16,), jnp.float32)

        def zb(i, _):
            zbuf[pl.ds(i * 16, 16)] = zero16
            return 0

        lax.fori_loop(0, RPT // 16, zb, 0)
        for i in range(CHUNK // 16):
            ones[pl.ds(i * 16, 16)] = one16
        base = s * RPT
        pltpu.sync_copy(zbuf, acc.at[pl.ds(base, RPT)])
        pltpu.sync_copy(eidx_hbm.at[1, wid], idx_d)
        plsc.subcore_barrier()

        def body(j, _):
            pltpu.sync_copy(ones, acc.at[idx_d.at[j]], add=True)
            return 0

        lax.fori_loop(0, NCH, body, 0)
        plsc.subcore_barrier()
        pltpu.sync_copy(acc.at[pl.ds(base, RPT)],
                        out_hbm.at[pl.ds(c * NP + base, RPT)])

    return deg_kernel


def _sc_segsum(W):
    mesh = plsc.VectorSubcoreMesh(core_axis_name="c", subcore_axis_name="s")

    @functools.partial(
        pl.kernel,
        compiler_params=pltpu.CompilerParams(use_tc_tiling_on_sc=False),
        out_type=jax.ShapeDtypeStruct((NCORES, NP, W), jnp.float32),
        mesh=mesh,
        scratch_types=[
            pltpu.VMEM_SHARED((NP, W), jnp.float32),
            pltpu.VMEM((NCH, CHUNK), jnp.int32),
            pltpu.VMEM((NCH, CHUNK), jnp.int32),
            pltpu.VMEM((2, CHUNK, W), jnp.float32),
            pltpu.SemaphoreType.DMA,
            pltpu.SemaphoreType.DMA,
            pltpu.SemaphoreType.DMA,
            pltpu.SemaphoreType.DMA,
        ],
    )
    def seg_kernel(y_hbm, eidx_hbm, out_hbm, acc, idx_s, idx_d, rows,
                   sg0, sg1, st0, st1):
        c = lax.axis_index("c")
        s = lax.axis_index("s")
        wid = s * NCORES + c
        zero16 = jnp.zeros((16,), jnp.float32)
        r0 = rows.at[0]
        r1 = rows.at[1]

        def zb(r, _):
            for k in range(W // 16):
                rows[0, r, pl.ds(k * 16, 16)] = zero16
            return 0

        lax.fori_loop(0, CHUNK, zb, 0)
        base = s * RPT
        off = 0
        while off < RPT:
            step = min(CHUNK, RPT - off)
            pltpu.sync_copy(rows.at[0, pl.ds(0, step)],
                            acc.at[pl.ds(base + off, step)])
            off += step
        pltpu.sync_copy(eidx_hbm.at[0, wid], idx_s)
        pltpu.sync_copy(eidx_hbm.at[1, wid], idx_d)
        plsc.subcore_barrier()

        pltpu.async_copy(y_hbm.at[idx_s.at[0]], r0, sg0)

        def body(i, _):
            a = 2 * i
            b = a + 1
            pltpu.make_async_copy(y_hbm.at[idx_s.at[a]], r0, sg0).wait()
            pltpu.async_copy(r0, acc.at[idx_d.at[a]], st0, add=True)

            @pl.when(i > 0)
            def _():
                pltpu.make_async_copy(r1, acc.at[idx_d.at[a]], st1).wait()

            pltpu.async_copy(y_hbm.at[idx_s.at[b]], r1, sg1)
            pltpu.make_async_copy(r0, acc.at[idx_d.at[a]], st0).wait()
            pltpu.make_async_copy(y_hbm.at[idx_s.at[b]], r1, sg1).wait()
            pltpu.async_copy(r1, acc.at[idx_d.at[b]], st1, add=True)
            pltpu.async_copy(y_hbm.at[idx_s.at[a + 2]], r0, sg0)
            return 0

        lax.fori_loop(0, (NCH - 1) // 2, body, 0)
        pltpu.make_async_copy(r1, acc.at[idx_d.at[0]], st1).wait()
        pltpu.make_async_copy(y_hbm.at[idx_s.at[NCH - 1]], r0, sg0).wait()
        pltpu.sync_copy(r0, acc.at[idx_d.at[NCH - 1]], add=True)
        plsc.subcore_barrier()
        pltpu.sync_copy(acc.at[pl.ds(base, RPT)], out_hbm.at[c, pl.ds(base, RPT)])

    return seg_kernel


_DEG = _sc_degree()
_SEG128 = _sc_segsum(D)
_SEG16 = _sc_segsum(16)


def _dis(d0, d1):
    return lax.rsqrt(d0 + d1 + 1.0)


def _inorm_relu(t):
    m = jnp.mean(t, axis=-1, keepdims=True)
    v = jnp.mean((t - m) * (t - m), axis=-1, keepdims=True)
    return jnp.maximum((t - m) / jnp.sqrt(v + 1e-5), 0.0)


def _rowmask(i, val):
    rows = i * BR + lax.broadcasted_iota(jnp.int32, (BR, 1), 0)
    return jnp.where(rows < N, val, 0.0)


def _tc1(x, W1, d0, d1):
    def body(x_ref, w_ref, d0_ref, d1_ref, y_ref):
        i = pl.program_id(0)
        dis = _dis(d0_ref[...], d1_ref[...])
        y = jnp.dot(x_ref[...], w_ref[...],
                    preferred_element_type=jnp.float32) * dis
        y_ref[...] = _rowmask(i, y)

    return pl.pallas_call(
        body,
        grid=(NP // BR,),
        in_specs=[
            pl.BlockSpec((BR, D), lambda i: (i, 0)),
            pl.BlockSpec((D, D), lambda i: (0, 0)),
            pl.BlockSpec((BR, 1), lambda i: (i, 0)),
            pl.BlockSpec((BR, 1), lambda i: (i, 0)),
        ],
        out_specs=pl.BlockSpec((BR, D), lambda i: (i, 0)),
        out_shape=jax.ShapeDtypeStruct((NP, D), jnp.float32),
    )(x, W1, d0, d1)


def _tc2(p, y1, d0, d1, b1, W2):
    def body(p_ref, y_ref, d0_ref, d1_ref, b_ref, w_ref, o_ref):
        i = pl.program_id(0)
        dis = _dis(d0_ref[...], d1_ref[...])
        t = dis * (p_ref[0] + p_ref[1] + y_ref[...]) + b_ref[...]
        h = _inorm_relu(t)
        y = jnp.dot(h, w_ref[...], preferred_element_type=jnp.float32) * dis
        o_ref[...] = _rowmask(i, y)

    return pl.pallas_call(
        body,
        grid=(NP // BR,),
        in_specs=[
            pl.BlockSpec((2, BR, D), lambda i: (0, i, 0)),
            pl.BlockSpec((BR, D), lambda i: (i, 0)),
            pl.BlockSpec((BR, 1), lambda i: (i, 0)),
            pl.BlockSpec((BR, 1), lambda i: (i, 0)),
            pl.BlockSpec((1, D), lambda i: (0, 0)),
            pl.BlockSpec((D, D), lambda i: (0, 0)),
        ],
        out_specs=pl.BlockSpec((BR, D), lambda i: (i, 0)),
        out_shape=jax.ShapeDtypeStruct((NP, D), jnp.float32),
    )(p, y1, d0, d1, b1, W2)


def _tc3(p, y2, d0, d1, b2, x, Wres, bres, Woutp):
    def body(p_ref, y_ref, d0_ref, d1_ref, b_ref, x_ref, wr_ref,
             br_ref, wo_ref, xo_ref, z_ref):
        i = pl.program_id(0)
        dis = _dis(d0_ref[...], d1_ref[...])
        t = dis * (p_ref[0] + p_ref[1] + y_ref[...]) + b_ref[...]
        h = _inorm_relu(t)
        xo = h + jnp.dot(x_ref[...], wr_ref[...],
                         preferred_element_type=jnp.float32) + br_ref[...]
        xo_ref[...] = xo
        z = jnp.dot(xo, wo_ref[...], preferred_element_type=jnp.float32) * dis
        z_ref[...] = _rowmask(i, z)

    return pl.pallas_call(
        body,
        grid=(NP // BR,),
        in_specs=[
            pl.BlockSpec((2, BR, D), lambda i: (0, i, 0)),
            pl.BlockSpec((BR, D), lambda i: (i, 0)),
            pl.BlockSpec((BR, 1), lambda i: (i, 0)),
            pl.BlockSpec((BR, 1), lambda i: (i, 0)),
            pl.BlockSpec((1, D), lambda i: (0, 0)),
            pl.BlockSpec((BR, D), lambda i: (i, 0)),
            pl.BlockSpec((D, D), lambda i: (0, 0)),
            pl.BlockSpec((1, D), lambda i: (0, 0)),
            pl.BlockSpec((D, 16), lambda i: (0, 0)),
        ],
        out_specs=[
            pl.BlockSpec((BR, D), lambda i: (i, 0)),
            pl.BlockSpec((BR, 16), lambda i: (i, 0)),
        ],
        out_shape=[
            jax.ShapeDtypeStruct((NP, D), jnp.float32),
            jax.ShapeDtypeStruct((NP, 16), jnp.float32),
        ],
    )(p, y2, d0, d1, b2, x, Wres, bres, Woutp)


def _tc4(q, zp, d0, d1, boutp):
    def body(q_ref, z_ref, d0_ref, d1_ref, b_ref, o_ref):
        dis = _dis(d0_ref[...], d1_ref[...])
        t = dis * (q_ref[0] + q_ref[1] + z_ref[...]) + b_ref[...]
        o_ref[...] = jax.nn.sigmoid(t) - 0.5

    return pl.pallas_call(
        body,
        grid=(NP // BR,),
        in_specs=[
            pl.BlockSpec((2, BR, 16), lambda i: (0, i, 0)),
            pl.BlockSpec((BR, 16), lambda i: (i, 0)),
            pl.BlockSpec((BR, 1), lambda i: (i, 0)),
            pl.BlockSpec((BR, 1), lambda i: (i, 0)),
            pl.BlockSpec((1, 16), lambda i: (0, 0)),
        ],
        out_specs=pl.BlockSpec((BR, 16), lambda i: (i, 0)),
        out_shape=jax.ShapeDtypeStruct((NP, 16), jnp.float32),
    )(q, zp, d0, d1, boutp)


def kernel(x, edge_index, W1, b1, W2, b2, Wres, bres, Wout, bout):
    eidx = edge_index.reshape(2, NW, EPT)
    eidx = jnp.pad(eidx, ((0, 0), (0, 0), (0, PADE - EPT)),
                   constant_values=PADV)
    eidx = eidx.reshape(2, NW, NCH, CHUNK)

    degp = _DEG(eidx)
    d0 = degp[:NP].reshape(NP, 1)
    d1 = degp[NP:].reshape(NP, 1)

    y1 = _tc1(x, W1, d0, d1)
    s1 = _SEG128(y1, eidx)
    y2 = _tc2(s1, y1, d0, d1, b1.reshape(1, D), W2)
    s2 = _SEG128(y2, eidx)

    Woutp = jnp.zeros((D, 16), jnp.float32).at[:, :2].set(Wout)
    x_out_p, zp = _tc3(s2, y2, d0, d1, b2.reshape(1, D), x,
                       Wres, bres.reshape(1, D), Woutp)
    s3 = _SEG16(zp, eidx)
    boutp = jnp.zeros((1, 16), jnp.float32).at[0, :2].set(bout)
    out_p = _tc4(s3, zp, d0, d1, boutp)
    return x_out_p[:N], out_p[:N, :2]


# speedup vs baseline: 26.1005x; 2.0348x over previous
import functools

import jax
import jax.numpy as jnp
from jax import lax
from jax.experimental import pallas as pl
from jax.experimental.pallas import tpu as pltpu
from jax.experimental.pallas import tpu_sc as plsc

N = 10000
NP = 10240
E = 320000
D = 128
NCORES = 2
NSUB = 16
NW = NCORES * NSUB
EPT = E // NW
CHUNK = 112
NCH = 91
PADE = NCH * CHUNK
PADV = NP - 1
RPT = NP // NSUB
BR = 1024


def _sc_degree():
    mesh = plsc.VectorSubcoreMesh(core_axis_name="c", subcore_axis_name="s")

    @functools.partial(
        pl.kernel,
        compiler_params=pltpu.CompilerParams(use_tc_tiling_on_sc=False),
        out_type=jax.ShapeDtypeStruct((NCORES * NP,), jnp.float32),
        mesh=mesh,
        scratch_types=[
            pltpu.VMEM_SHARED((NP,), jnp.float32),
            pltpu.VMEM((NCH, CHUNK), jnp.int32),
            pltpu.VMEM((RPT,), jnp.float32),
            pltpu.VMEM((CHUNK,), jnp.float32),
        ],
    )
    def deg_kernel(eidx_hbm, out_hbm, acc, idx_d, zbuf, ones):
        c = lax.axis_index("c")
        s = lax.axis_index("s")
        wid = s * NCORES + c
        zero16 = jnp.zeros((16,), jnp.float32)
        one16 = jnp.ones((16,), jnp.float32)

        def zb(i, _):
            zbuf[pl.ds(i * 16, 16)] = zero16
            return 0

        lax.fori_loop(0, RPT // 16, zb, 0)
        for i in range(CHUNK // 16):
            ones[pl.ds(i * 16, 16)] = one16
        base = s * RPT
        pltpu.sync_copy(zbuf, acc.at[pl.ds(base, RPT)])
        pltpu.sync_copy(eidx_hbm.at[1, wid], idx_d)
        plsc.subcore_barrier()

        def body(j, _):
            pltpu.sync_copy(ones, acc.at[idx_d.at[j]], add=True)
            return 0

        lax.fori_loop(0, NCH, body, 0)
        plsc.subcore_barrier()
        pltpu.sync_copy(acc.at[pl.ds(base, RPT)],
                        out_hbm.at[pl.ds(c * NP + base, RPT)])

    return deg_kernel


def _sc_segsum(W):
    mesh = plsc.VectorSubcoreMesh(core_axis_name="c", subcore_axis_name="s")

    @functools.partial(
        pl.kernel,
        compiler_params=pltpu.CompilerParams(use_tc_tiling_on_sc=False),
        out_type=jax.ShapeDtypeStruct((NCORES, NP, W), jnp.float32),
        mesh=mesh,
        scratch_types=[
            pltpu.VMEM_SHARED((NP, W), jnp.float32),
            pltpu.VMEM((NCH, CHUNK), jnp.int32),
            pltpu.VMEM((NCH, CHUNK), jnp.int32),
            pltpu.VMEM((2, CHUNK, W), jnp.float32),
            pltpu.SemaphoreType.DMA,
            pltpu.SemaphoreType.DMA,
            pltpu.SemaphoreType.DMA,
            pltpu.SemaphoreType.DMA,
        ],
    )
    def seg_kernel(y_hbm, eidx_hbm, out_hbm, acc, idx_s, idx_d, rows,
                   sg0, sg1, st0, st1):
        c = lax.axis_index("c")
        s = lax.axis_index("s")
        wid = s * NCORES + c
        zero16 = jnp.zeros((16,), jnp.float32)
        r0 = rows.at[0]
        r1 = rows.at[1]

        def zb(r, _):
            for k in range(W // 16):
                rows[0, r, pl.ds(k * 16, 16)] = zero16
            return 0

        lax.fori_loop(0, CHUNK, zb, 0)
        base = s * RPT
        off = 0
        while off < RPT:
            step = min(CHUNK, RPT - off)
            pltpu.sync_copy(rows.at[0, pl.ds(0, step)],
                            acc.at[pl.ds(base + off, step)])
            off += step
        pltpu.sync_copy(eidx_hbm.at[0, wid], idx_s)
        pltpu.sync_copy(eidx_hbm.at[1, wid], idx_d)
        plsc.subcore_barrier()

        pltpu.async_copy(y_hbm.at[idx_s.at[0]], r0, sg0)

        def body(i, _):
            a = 2 * i
            b = a + 1
            pltpu.make_async_copy(y_hbm.at[idx_s.at[a]], r0, sg0).wait()
            pltpu.async_copy(r0, acc.at[idx_d.at[a]], st0, add=True)

            @pl.when(i > 0)
            def _():
                pltpu.make_async_copy(r1, acc.at[idx_d.at[a]], st1).wait()

            pltpu.async_copy(y_hbm.at[idx_s.at[b]], r1, sg1)
            pltpu.make_async_copy(r0, acc.at[idx_d.at[a]], st0).wait()
            pltpu.make_async_copy(y_hbm.at[idx_s.at[b]], r1, sg1).wait()
            pltpu.async_copy(r1, acc.at[idx_d.at[b]], st1, add=True)
            pltpu.async_copy(y_hbm.at[idx_s.at[a + 2]], r0, sg0)
            return 0

        lax.fori_loop(0, (NCH - 1) // 2, body, 0)
        pltpu.make_async_copy(r1, acc.at[idx_d.at[0]], st1).wait()
        pltpu.make_async_copy(y_hbm.at[idx_s.at[NCH - 1]], r0, sg0).wait()
        pltpu.sync_copy(r0, acc.at[idx_d.at[NCH - 1]], add=True)
        plsc.subcore_barrier()
        pltpu.sync_copy(acc.at[pl.ds(base, RPT)], out_hbm.at[c, pl.ds(base, RPT)])

    return seg_kernel


_DEG = _sc_degree()
_SEG128 = _sc_segsum(D)
_SEG16 = _sc_segsum(16)


def _dis(d0, d1):
    return lax.rsqrt(d0 + d1 + 1.0)


def _inorm_relu(t):
    m = jnp.mean(t, axis=-1, keepdims=True)
    v = jnp.mean((t - m) * (t - m), axis=-1, keepdims=True)
    return jnp.maximum((t - m) / jnp.sqrt(v + 1e-5), 0.0)


def _rowmask(i, val):
    rows = i * BR + lax.broadcasted_iota(jnp.int32, (BR, 1), 0)
    return jnp.where(rows < N, val, 0.0)


def _tc1(x, W1, d0, d1):
    def body(x_ref, w_ref, d0_ref, d1_ref, y_ref):
        i = pl.program_id(0)
        dis = _dis(d0_ref[...], d1_ref[...])
        y = jnp.dot(x_ref[...], w_ref[...],
                    preferred_element_type=jnp.float32) * dis
        y_ref[...] = _rowmask(i, y)

    return pl.pallas_call(
        body,
        grid=(NP // BR,),
        in_specs=[
            pl.BlockSpec((BR, D), lambda i: (i, 0)),
            pl.BlockSpec((D, D), lambda i: (0, 0)),
            pl.BlockSpec((BR, 1), lambda i: (i, 0)),
            pl.BlockSpec((BR, 1), lambda i: (i, 0)),
        ],
        out_specs=pl.BlockSpec((BR, D), lambda i: (i, 0)),
        out_shape=jax.ShapeDtypeStruct((NP, D), jnp.float32),
    )(x, W1, d0, d1)


def _tc2(p, y1, d0, d1, b1, W2):
    def body(p_ref, y_ref, d0_ref, d1_ref, b_ref, w_ref, o_ref):
        i = pl.program_id(0)
        dis = _dis(d0_ref[...], d1_ref[...])
        t = dis * (p_ref[0] + p_ref[1] + y_ref[...]) + b_ref[...]
        h = _inorm_relu(t)
        y = jnp.dot(h, w_ref[...], preferred_element_type=jnp.float32) * dis
        o_ref[...] = _rowmask(i, y)

    return pl.pallas_call(
        body,
        grid=(NP // BR,),
        in_specs=[
            pl.BlockSpec((2, BR, D), lambda i: (0, i, 0)),
            pl.BlockSpec((BR, D), lambda i: (i, 0)),
            pl.BlockSpec((BR, 1), lambda i: (i, 0)),
            pl.BlockSpec((BR, 1), lambda i: (i, 0)),
            pl.BlockSpec((1, D), lambda i: (0, 0)),
            pl.BlockSpec((D, D), lambda i: (0, 0)),
        ],
        out_specs=pl.BlockSpec((BR, D), lambda i: (i, 0)),
        out_shape=jax.ShapeDtypeStruct((NP, D), jnp.float32),
    )(p, y1, d0, d1, b1, W2)


def _tc3(p, y2, d0, d1, b2, x, Wres, bres, Woutp):
    def body(p_ref, y_ref, d0_ref, d1_ref, b_ref, x_ref, wr_ref,
             br_ref, wo_ref, xo_ref, z_ref):
        i = pl.program_id(0)
        dis = _dis(d0_ref[...], d1_ref[...])
        t = dis * (p_ref[0] + p_ref[1] + y_ref[...]) + b_ref[...]
        h = _inorm_relu(t)
        xo = h + jnp.dot(x_ref[...], wr_ref[...],
                         preferred_element_type=jnp.float32) + br_ref[...]
        xo_ref[...] = xo
        z = jnp.dot(xo, wo_ref[...], preferred_element_type=jnp.float32) * dis
        z_ref[...] = _rowmask(i, z)

    return pl.pallas_call(
        body,
        grid=(NP // BR,),
        in_specs=[
            pl.BlockSpec((2, BR, D), lambda i: (0, i, 0)),
            pl.BlockSpec((BR, D), lambda i: (i, 0)),
            pl.BlockSpec((BR, 1), lambda i: (i, 0)),
            pl.BlockSpec((BR, 1), lambda i: (i, 0)),
            pl.BlockSpec((1, D), lambda i: (0, 0)),
            pl.BlockSpec((BR, D), lambda i: (i, 0)),
            pl.BlockSpec((D, D), lambda i: (0, 0)),
            pl.BlockSpec((1, D), lambda i: (0, 0)),
            pl.BlockSpec((D, 16), lambda i: (0, 0)),
        ],
        out_specs=[
            pl.BlockSpec((BR, D), lambda i: (i, 0)),
            pl.BlockSpec((BR, 16), lambda i: (i, 0)),
        ],
        out_shape=[
            jax.ShapeDtypeStruct((NP, D), jnp.float32),
            jax.ShapeDtypeStruct((NP, 16), jnp.float32),
        ],
    )(p, y2, d0, d1, b2, x, Wres, bres, Woutp)


def _tc4(q, zp, d0, d1, boutp):
    def body(q_ref, z_ref, d0_ref, d1_ref, b_ref, o_ref):
        dis = _dis(d0_ref[...], d1_ref[...])
        t = dis * (q_ref[0] + q_ref[1] + z_ref[...]) + b_ref[...]
        o_ref[...] = jax.nn.sigmoid(t) - 0.5

    return pl.pallas_call(
        body,
        grid=(NP // BR,),
        in_specs=[
            pl.BlockSpec((2, BR, 16), lambda i: (0, i, 0)),
            pl.BlockSpec((BR, 16), lambda i: (i, 0)),
            pl.BlockSpec((BR, 1), lambda i: (i, 0)),
            pl.BlockSpec((BR, 1), lambda i: (i, 0)),
            pl.BlockSpec((1, 16), lambda i: (0, 0)),
        ],
        out_specs=pl.BlockSpec((BR, 16), lambda i: (i, 0)),
        out_shape=jax.ShapeDtypeStruct((NP, 16), jnp.float32),
    )(q, zp, d0, d1, boutp)


def kernel(x, edge_index, W1, b1, W2, b2, Wres, bres, Wout, bout):
    npad = PADE - EPT
    spread = (jnp.arange(npad, dtype=jnp.int32)[None, :]
              + jnp.arange(NW, dtype=jnp.int32)[:, None] * 8) % (NP - N) + N
    pads = jnp.broadcast_to(spread[None], (2, NW, npad))
    eidx = jnp.concatenate([edge_index.reshape(2, NW, EPT), pads], axis=2)
    eidx = eidx.reshape(2, NW, NCH, CHUNK)

    degp = _DEG(eidx)
    d0 = degp[:NP].reshape(NP, 1)
    d1 = degp[NP:].reshape(NP, 1)

    y1 = _tc1(x, W1, d0, d1)
    s1 = _SEG128(y1, eidx)
    y2 = _tc2(s1, y1, d0, d1, b1.reshape(1, D), W2)
    s2 = _SEG128(y2, eidx)

    Woutp = jnp.zeros((D, 16), jnp.float32).at[:, :2].set(Wout)
    x_out_p, zp = _tc3(s2, y2, d0, d1, b2.reshape(1, D), x,
                       Wres, bres.reshape(1, D), Woutp)
    s3 = _SEG16(zp, eidx)
    boutp = jnp.zeros((1, 16), jnp.float32).at[0, :2].set(bout)
    out_p = _tc4(s3, zp, d0, d1, boutp)
    return x_out_p[:N], out_p[:N, :2]
